# baseline (speedup 1.0000x reference)
import jax
import jax.numpy as jnp
from jax.experimental import pallas as pl

_BLK = 512


def _gating_body(x_ref, gw_ref, gb_ref, out_ref, aux_ref):
    xb = x_ref[...]
    out_ref[...] = xb

    logits = jnp.dot(xb, gw_ref[...], preferred_element_type=jnp.float32)
    logits = logits + gb_ref[0, :]

    m1 = jnp.max(logits, axis=-1, keepdims=True)
    i1 = jnp.argmax(logits, axis=-1)
    col = jax.lax.broadcasted_iota(jnp.int32, logits.shape, 1)
    masked = jnp.where(col == i1[:, None], -1e30, logits)
    m2 = jnp.max(masked, axis=-1, keepdims=True)
    i2 = jnp.argmax(masked, axis=-1)

    e = jnp.exp(m2 - m1)
    denom = 1.0 + e
    p1 = 1.0 / denom
    p2 = e / denom

    aux_ref[...] = jnp.concatenate(
        [p1, p2, i1[:, None].astype(jnp.float32), i2[:, None].astype(jnp.float32)],
        axis=1,
    )


def kernel(x, gate_w, gate_b, w1, b1, w2, b2):
    bsz, n, dim = x.shape
    e = gate_w.shape[1]
    rows = bsz * n
    xf = x.reshape(rows, dim)
    gb2 = gate_b.reshape(1, e)

    grid = rows // _BLK
    out, _aux = pl.pallas_call(
        _gating_body,
        grid=(grid,),
        in_specs=[
            pl.BlockSpec((_BLK, dim), lambda i: (i, 0)),
            pl.BlockSpec((dim, e), lambda i: (0, 0)),
            pl.BlockSpec((1, e), lambda i: (0, 0)),
        ],
        out_specs=[
            pl.BlockSpec((_BLK, dim), lambda i: (i, 0)),
            pl.BlockSpec((_BLK, 4), lambda i: (i, 0)),
        ],
        out_shape=[
            jax.ShapeDtypeStruct((rows, dim), x.dtype),
            jax.ShapeDtypeStruct((rows, 4), jnp.float32),
        ],
    )(xf, gate_w, gb2)
    return out.reshape(bsz, n, dim)


# speedup vs baseline: 1.1266x; 1.1266x over previous
import jax
import jax.numpy as jnp
from jax.experimental import pallas as pl
from jax.experimental.pallas import tpu as pltpu

_BLK = 1024


def _gating_body(x_ref, gw_ref, gb_ref, out_ref, aux_ref):
    xb = x_ref[...]
    out_ref[...] = xb

    logits = jnp.dot(xb, gw_ref[...], preferred_element_type=jnp.float32)
    logits = logits + gb_ref[0, :]

    m1 = jnp.max(logits, axis=-1, keepdims=True)
    i1 = jnp.argmax(logits, axis=-1)
    col = jax.lax.broadcasted_iota(jnp.int32, logits.shape, 1)
    masked = jnp.where(col == i1[:, None], -1e30, logits)
    m2 = jnp.max(masked, axis=-1, keepdims=True)
    i2 = jnp.argmax(masked, axis=-1)

    e = jnp.exp(m2 - m1)
    denom = 1.0 + e
    p1 = 1.0 / denom
    p2 = e / denom

    aux_ref[...] = jnp.concatenate(
        [p1, p2, i1[:, None].astype(jnp.float32), i2[:, None].astype(jnp.float32)],
        axis=1,
    )


def kernel(x, gate_w, gate_b, w1, b1, w2, b2):
    bsz, n, dim = x.shape
    e = gate_w.shape[1]
    rows = bsz * n
    xf = x.reshape(rows, dim)
    gb2 = gate_b.reshape(1, e)

    grid = rows // _BLK
    out, _aux = pl.pallas_call(
        _gating_body,
        grid=(grid,),
        in_specs=[
            pl.BlockSpec((_BLK, dim), lambda i: (i, 0)),
            pl.BlockSpec((dim, e), lambda i: (0, 0)),
            pl.BlockSpec((1, e), lambda i: (0, 0)),
        ],
        out_specs=[
            pl.BlockSpec((_BLK, dim), lambda i: (i, 0)),
            pl.BlockSpec((_BLK, 4), lambda i: (i, 0)),
        ],
        out_shape=[
            jax.ShapeDtypeStruct((rows, dim), x.dtype),
            jax.ShapeDtypeStruct((rows, 4), jnp.float32),
        ],
        compiler_params=pltpu.CompilerParams(
            dimension_semantics=("parallel",),
        ),
    )(xf, gate_w, gb2)
    return out.reshape(bsz, n, dim)
